# baseline (speedup 1.0000x reference)
import dataclasses
import functools

import jax
import jax.numpy as jnp
from jax import lax
from jax.experimental import pallas as pl
from jax.experimental.pallas import tpu as pltpu
from jax.experimental.pallas import tpu_sc as plsc

NBLK = 8192
NTILES = 32
LANES = 16


def _tc_body(spikes_ref, ratesT_ref, out_ref):
    i = pl.program_id(0)

    r = ratesT_ref[...]
    k = r.shape[0]

    norm = jnp.sum(jnp.abs(r), axis=0, keepdims=True)
    mx = jnp.max(r, axis=0, keepdims=True)
    sub = jax.lax.broadcasted_iota(jnp.int32, r.shape, 0)
    idx = jnp.min(jnp.where(r == mx, sub, k), axis=0, keepdims=True)
    w = mx / jnp.maximum(norm, 1e-12)

    oh = jnp.where(sub == idx, w, 0.0).astype(jnp.bfloat16)

    part = jax.lax.dot_general(
        spikes_ref[...].astype(jnp.bfloat16), oh, (((1,), (1,)), ((), ())),
        preferred_element_type=jnp.float32)

    @pl.when(i == 0)
    def _():
        out_ref[...] = jnp.zeros_like(out_ref)

    out_ref[...] += part


def _tc_logits(spikes, ratesT):
    b, n = spikes.shape
    k = ratesT.shape[0]
    return pl.pallas_call(
        _tc_body,
        grid=(n // NBLK,),
        in_specs=[
            pl.BlockSpec((b, NBLK), lambda i: (0, i)),
            pl.BlockSpec((k, NBLK), lambda i: (0, i)),
        ],
        out_specs=pl.BlockSpec((b, k), lambda i: (0, 0)),
        out_shape=jax.ShapeDtypeStruct((b, k), jnp.float32),
        compiler_params=pltpu.CompilerParams(
            dimension_semantics=("arbitrary",),
        ),
    )(spikes, ratesT)


def _sc_bincount(rates):
    n, k = rates.shape
    ch = n // NTILES
    mesh = plsc.VectorSubcoreMesh(core_axis_name="c", subcore_axis_name="s")
    cp = pltpu.CompilerParams()
    if "needs_layout_passes" in pltpu.CompilerParams.__dataclass_fields__:
        cp = dataclasses.replace(cp, needs_layout_passes=False)

    @functools.partial(
        pl.kernel,
        mesh=mesh,
        compiler_params=cp,
        out_type=jax.ShapeDtypeStruct((NTILES, k * LANES), jnp.float32),
        scratch_types=[
            pltpu.VMEM((ch * k,), jnp.float32),
            pltpu.VMEM((k * LANES,), jnp.float32),
        ],
    )
    def sc_kernel(rates_hbm, out_hbm, chunk_v, hist_v):
        wid = lax.axis_index("s") * 2 + lax.axis_index("c")
        pltpu.sync_copy(rates_hbm.at[pl.ds(wid * (ch * k), ch * k)], chunk_v)

        lane_iota = lax.broadcasted_iota(jnp.int32, (LANES,), 0)

        def group(g, hist):
            ib = lane_iota * k + g * (LANES * k)
            m = plsc.load_gather(chunk_v, [ib])
            a = jnp.zeros((LANES,), jnp.int32)
            for kk in range(1, k):
                v = plsc.load_gather(chunk_v, [ib + kk])
                gt = v > m
                m = jnp.where(gt, v, m)
                a = jnp.where(gt, kk, a)
            return tuple(
                hist[kk] + jnp.where(a == kk, 1.0, 0.0) for kk in range(k))

        hist0 = tuple(jnp.zeros((LANES,), jnp.float32) for _ in range(k))
        hist = lax.fori_loop(0, ch // LANES, group, hist0)
        for kk in range(k):
            hist_v[pl.ds(kk * LANES, LANES)] = hist[kk]
        pltpu.sync_copy(hist_v, out_hbm.at[wid])

    return sc_kernel(rates.reshape(-1))


def kernel(spikes, rates):
    ratesT = rates.T
    raw = _tc_logits(spikes, ratesT)
    partials = _sc_bincount(rates)
    k = rates.shape[1]
    counts = jnp.sum(partials.reshape(NTILES, k, LANES), axis=(0, 2))
    return jnp.where(counts > 0.0, raw / counts, 0.0)


# speedup vs baseline: 3.4860x; 3.4860x over previous
import jax
import jax.numpy as jnp
from jax.experimental import pallas as pl
from jax.experimental.pallas import tpu as pltpu

NBLK = 8192


def _fused_body(spikes_ref, ratesT_ref, out_ref, cnt_ref):
    i = pl.program_id(0)

    r = ratesT_ref[...]
    k = r.shape[0]

    norm = jnp.sum(jnp.abs(r), axis=0, keepdims=True)
    mx = jnp.max(r, axis=0, keepdims=True)
    sub = jax.lax.broadcasted_iota(jnp.int32, r.shape, 0)
    idx = jnp.min(jnp.where(r == mx, sub, k), axis=0, keepdims=True)
    w = mx / jnp.maximum(norm, 1e-12)

    hit = sub == idx
    oh = jnp.where(hit, w, 0.0).astype(jnp.bfloat16)
    ohc = jnp.where(hit, 1.0, 0.0).astype(jnp.bfloat16)

    part = jax.lax.dot_general(
        spikes_ref[...].astype(jnp.bfloat16), oh, (((1,), (1,)), ((), ())),
        preferred_element_type=jnp.float32)
    ones = jnp.ones((8, NBLK), jnp.bfloat16)
    cpart = jax.lax.dot_general(
        ones, ohc, (((1,), (1,)), ((), ())),
        preferred_element_type=jnp.float32)

    @pl.when(i == 0)
    def _():
        out_ref[...] = jnp.zeros_like(out_ref)
        cnt_ref[...] = jnp.zeros_like(cnt_ref)

    out_ref[...] += part
    cnt_ref[:, 0:cpart.shape[1]] += cpart

    @pl.when(i == pl.num_programs(0) - 1)
    def _():
        cnt = cnt_ref[0:1, 0:out_ref.shape[1]]
        acc = out_ref[...]
        out_ref[...] = jnp.where(cnt > 0.0, acc / cnt, 0.0)


def kernel(spikes, rates):
    b, n = spikes.shape
    k = rates.shape[1]

    ratesT = rates.T

    return pl.pallas_call(
        _fused_body,
        grid=(n // NBLK,),
        in_specs=[
            pl.BlockSpec((b, NBLK), lambda i: (0, i)),
            pl.BlockSpec((k, NBLK), lambda i: (0, i)),
        ],
        out_specs=pl.BlockSpec((b, k), lambda i: (0, 0)),
        out_shape=jax.ShapeDtypeStruct((b, k), jnp.float32),
        scratch_shapes=[pltpu.VMEM((8, 16), jnp.float32)],
        compiler_params=pltpu.CompilerParams(
            dimension_semantics=("arbitrary",),
        ),
    )(spikes, ratesT)
